# Initial kernel scaffold; baseline (speedup 1.0000x reference)
import jax
import jax.numpy as jnp
from jax.experimental import pallas as pl


def kernel(positions, embeddings):
    raise NotImplementedError("write your pallas kernel here")


# speedup vs baseline: 2.1479x; 2.1479x over previous
import functools

import jax
import jax.numpy as jnp
from jax import lax
from jax.experimental import pallas as pl
from jax.experimental.pallas import tpu as pltpu
from jax.experimental.pallas import tpu_sc as plsc

GX = 512
GY = 512
F = 64
B = 16384
N = 200
P = B * N
NC, NS = 2, 16
NW = NC * NS
PPW = P // NW
CHUNK = 128
NCHUNK = PPW // CHUNK
L = 16
GROUPS = CHUNK // L


def _sc_body(xs, ys, emb, out, x_v, y_v, dx_v, dy_v,
             i00_v, i01_v, i10_v, i11_v,
             r00_v, r01_v, r10_v, r11_v, out_v, sem):
    wid = lax.axis_index("s") * NC + lax.axis_index("c")
    base = wid * PPW

    def chunk_body(ci, _):
        off = base + ci * CHUNK
        pltpu.sync_copy(xs.at[pl.ds(off, CHUNK)], x_v)
        pltpu.sync_copy(ys.at[pl.ds(off, CHUNK)], y_v)

        for g in range(GROUPS):
            sl = pl.ds(g * L, L)
            x = x_v[sl] * float(GX)
            y = y_v[sl] * float(GY)
            x0 = x.astype(jnp.int32)
            y0 = y.astype(jnp.int32)
            dx_v[sl] = x - x0.astype(jnp.float32)
            dy_v[sl] = y - y0.astype(jnp.float32)
            x0c = jnp.clip(x0, 0, GX - 1)
            x1c = jnp.clip(x0 + 1, 0, GX - 1)
            y0c = jnp.clip(y0, 0, GY - 1)
            y1c = jnp.clip(y0 + 1, 0, GY - 1)
            r0 = x0c * GY
            r1 = x1c * GY
            i00_v[sl] = r0 + y0c
            i01_v[sl] = r0 + y1c
            i10_v[sl] = r1 + y0c
            i11_v[sl] = r1 + y1c

        c0 = pltpu.async_copy(emb.at[i00_v], r00_v, sem)
        c1 = pltpu.async_copy(emb.at[i01_v], r01_v, sem)
        c2 = pltpu.async_copy(emb.at[i10_v], r10_v, sem)
        c3 = pltpu.async_copy(emb.at[i11_v], r11_v, sem)
        c0.wait()
        c1.wait()
        c2.wait()
        c3.wait()

        def comb(p, _):
            pidx = jnp.full((L,), p, dtype=jnp.int32)
            dxb = plsc.load_gather(dx_v, [pidx])
            dyb = plsc.load_gather(dy_v, [pidx])
            for q in range(F // L):
                qs = pl.ds(q * L, L)
                f00 = r00_v[p, qs]
                f01 = r01_v[p, qs]
                f10 = r10_v[p, qs]
                f11 = r11_v[p, qs]
                t0 = f00 + dxb * (f10 - f00)
                t1 = f01 + dxb * (f11 - f01)
                out_v[p, qs] = t0 + dyb * (t1 - t0)
            return 0

        lax.fori_loop(0, CHUNK, comb, 0)
        pltpu.sync_copy(out_v, out.at[pl.ds(off, CHUNK)])
        return 0

    lax.fori_loop(0, NCHUNK, chunk_body, 0)


@jax.jit
def _bilinear_sc(xs, ys, emb):
    mesh = plsc.VectorSubcoreMesh(
        core_axis_name="c", subcore_axis_name="s",
        num_cores=NC, num_subcores=NS)
    f = functools.partial(
        pl.kernel,
        out_type=jax.ShapeDtypeStruct((P, F), jnp.float32),
        mesh=mesh,
        scratch_types=[
            pltpu.VMEM((CHUNK,), jnp.float32),
            pltpu.VMEM((CHUNK,), jnp.float32),
            pltpu.VMEM((CHUNK,), jnp.float32),
            pltpu.VMEM((CHUNK,), jnp.float32),
            pltpu.VMEM((CHUNK,), jnp.int32),
            pltpu.VMEM((CHUNK,), jnp.int32),
            pltpu.VMEM((CHUNK,), jnp.int32),
            pltpu.VMEM((CHUNK,), jnp.int32),
            pltpu.VMEM((CHUNK, F), jnp.float32),
            pltpu.VMEM((CHUNK, F), jnp.float32),
            pltpu.VMEM((CHUNK, F), jnp.float32),
            pltpu.VMEM((CHUNK, F), jnp.float32),
            pltpu.VMEM((CHUNK, F), jnp.float32),
            pltpu.SemaphoreType.DMA,
        ],
        compiler_params=pltpu.CompilerParams(
            use_tc_tiling_on_sc=False, needs_layout_passes=False),
    )(_sc_body)
    return f(xs, ys, emb)


def kernel(positions, embeddings):
    pos = positions.reshape(P, 2)
    xs = pos[:, 0]
    ys = pos[:, 1]
    emb = embeddings.reshape(GX * GY, F)
    out = _bilinear_sc(xs, ys, emb)
    return out.reshape(B, N, F)


# speedup vs baseline: 5.2285x; 2.4343x over previous
import functools

import jax
import jax.numpy as jnp
from jax import lax
from jax.experimental import pallas as pl
from jax.experimental.pallas import tpu as pltpu
from jax.experimental.pallas import tpu_sc as plsc

GX = 512
GY = 512
F = 64
B = 16384
N = 200
P = B * N
NC, NS = 2, 16
NW = NC * NS
PPW = P // NW
CHUNK = 128
NCHUNK = PPW // CHUNK
L = 16
GROUPS = CHUNK // L


def _sc_body(xs, ys, emb, out, bufs, sems):
    wid = lax.axis_index("s") * NC + lax.axis_index("c")
    base = wid * PPW
    sem_xy, sem_g, sem_o = sems

    def fire_xy(g, b):
        off = base + g * CHUNK
        x_v, y_v = bufs[b][0], bufs[b][1]
        pltpu.async_copy(xs.at[pl.ds(off, CHUNK)], x_v, sem_xy[b])
        pltpu.async_copy(ys.at[pl.ds(off, CHUNK)], y_v, sem_xy[b])

    def wait_xy(b):
        x_v, y_v = bufs[b][0], bufs[b][1]
        pltpu.make_async_copy(xs.at[pl.ds(0, CHUNK)], x_v, sem_xy[b]).wait()
        pltpu.make_async_copy(ys.at[pl.ds(0, CHUNK)], y_v, sem_xy[b]).wait()

    def index_phase(b):
        x_v, y_v, w_v, i_v = bufs[b][0], bufs[b][1], bufs[b][2], bufs[b][3]
        for g in range(GROUPS):
            sl = pl.ds(g * L, L)
            x = x_v[sl] * float(GX)
            y = y_v[sl] * float(GY)
            x0 = x.astype(jnp.int32)
            y0 = y.astype(jnp.int32)
            dx = x - x0.astype(jnp.float32)
            dy = y - y0.astype(jnp.float32)
            ex = 1.0 - dx
            ey = 1.0 - dy
            w_v[0][sl] = ex * ey
            w_v[1][sl] = ex * dy
            w_v[2][sl] = dx * ey
            w_v[3][sl] = dx * dy
            x0c = jnp.clip(x0, 0, GX - 1)
            x1c = jnp.clip(x0 + 1, 0, GX - 1)
            y0c = jnp.clip(y0, 0, GY - 1)
            y1c = jnp.clip(y0 + 1, 0, GY - 1)
            r0 = x0c * GY
            r1 = x1c * GY
            i_v[0][sl] = r0 + y0c
            i_v[1][sl] = r0 + y1c
            i_v[2][sl] = r1 + y0c
            i_v[3][sl] = r1 + y1c

    def fire_gathers(b):
        i_v, r_v = bufs[b][3], bufs[b][4]
        for k in range(4):
            pltpu.async_copy(emb.at[i_v[k]], r_v[k], sem_g[b])

    def wait_gathers(b):
        i_v, r_v = bufs[b][3], bufs[b][4]
        for k in range(4):
            pltpu.make_async_copy(emb.at[i_v[k]], r_v[k], sem_g[b]).wait()

    def combine(b):
        w_v, r_v, out_v = bufs[b][2], bufs[b][4], bufs[b][5]

        @plsc.parallel_loop(0, CHUNK, step=1, unroll=2)
        def comb(p):
            pidx = jnp.full((L,), p, dtype=jnp.int32)
            w00b = plsc.load_gather(w_v[0], [pidx])
            w01b = plsc.load_gather(w_v[1], [pidx])
            w10b = plsc.load_gather(w_v[2], [pidx])
            w11b = plsc.load_gather(w_v[3], [pidx])
            for q in range(F // L):
                qs = pl.ds(q * L, L)
                f00 = r_v[0][p, qs]
                f01 = r_v[1][p, qs]
                f10 = r_v[2][p, qs]
                f11 = r_v[3][p, qs]
                out_v[p, qs] = ((f00 * w00b + f01 * w01b)
                                + (f10 * w10b + f11 * w11b))

    def fire_out(g, b):
        off = base + g * CHUNK
        out_v = bufs[b][5]
        pltpu.async_copy(out_v, out.at[pl.ds(off, CHUNK)], sem_o[b])

    def wait_out(b):
        out_v = bufs[b][5]
        pltpu.make_async_copy(out_v, out.at[pl.ds(0, CHUNK)], sem_o[b]).wait()

    def step(g, b, first, last, prefetch=True):
        nb = 1 - b
        if not last:
            wait_xy(nb)
            index_phase(nb)
            fire_gathers(nb)
        if prefetch:
            fire_xy(g + 2, b)
        wait_gathers(b)
        combine(b)
        if not first:
            wait_out(b)
        fire_out(g, b)

    fire_xy(0, 0)
    fire_xy(1, 1)
    wait_xy(0)
    index_phase(0)
    fire_gathers(0)

    step(0, 0, first=True, last=False)
    step(1, 1, first=True, last=False)

    def pair(i, _):
        g = 2 + 2 * i
        step(g, 0, first=False, last=False)
        step(g + 1, 1, first=False, last=False)
        return 0

    lax.fori_loop(0, (NCHUNK - 4) // 2, pair, 0)

    step(NCHUNK - 2, 0, first=False, last=False, prefetch=False)
    step(NCHUNK - 1, 1, first=False, last=True, prefetch=False)
    wait_out(0)
    wait_out(1)


def _buf_spec():
    return (
        pltpu.VMEM((CHUNK,), jnp.float32),
        pltpu.VMEM((CHUNK,), jnp.float32),
        tuple(pltpu.VMEM((CHUNK,), jnp.float32) for _ in range(4)),
        tuple(pltpu.VMEM((CHUNK,), jnp.int32) for _ in range(4)),
        tuple(pltpu.VMEM((CHUNK, F), jnp.float32) for _ in range(4)),
        pltpu.VMEM((CHUNK, F), jnp.float32),
    )


@jax.jit
def _bilinear_sc(xs, ys, emb):
    mesh = plsc.VectorSubcoreMesh(
        core_axis_name="c", subcore_axis_name="s",
        num_cores=NC, num_subcores=NS)
    f = pl.kernel(
        _sc_body,
        out_type=jax.ShapeDtypeStruct((P, F), jnp.float32),
        mesh=mesh,
        scratch_types=[
            (_buf_spec(), _buf_spec()),
            (
                (pltpu.SemaphoreType.DMA, pltpu.SemaphoreType.DMA),
                (pltpu.SemaphoreType.DMA, pltpu.SemaphoreType.DMA),
                (pltpu.SemaphoreType.DMA, pltpu.SemaphoreType.DMA),
            ),
        ],
        compiler_params=pltpu.CompilerParams(
            use_tc_tiling_on_sc=False, needs_layout_passes=False),
    )
    return f(xs, ys, emb)


def kernel(positions, embeddings):
    pos = positions.reshape(P, 2)
    xs = pos[:, 0]
    ys = pos[:, 1]
    emb = embeddings.reshape(GX * GY, F)
    out = _bilinear_sc(xs, ys, emb)
    return out.reshape(B, N, F)
